# baseline (speedup 1.0000x reference)
import functools

import jax
import jax.numpy as jnp
from jax import lax
from jax.experimental import pallas as pl
from jax.experimental.pallas import tpu as pltpu
from jax.experimental.pallas import tpu_sc as plsc

BATCH = 16384
EMBED = 32
NC = 2
NS = 16
L = 16
NW = NC * NS
BPW = BATCH // NW
IDX_CHUNK = 128
NP = BPW // IDX_CHUNK
NCH = BPW // L


def _gmf_body(users_hbm, movies_hbm, ut_hbm, mt_hbm, wb_hbm, bb_hbm, out_hbm,
              uidx_v, midx_v, urows_v, mrows_v, w_v, b_v, o_v, sem):
    wid = lax.axis_index("s") * NC + lax.axis_index("c")
    base = wid * BPW

    pltpu.sync_copy(users_hbm.at[wid], uidx_v)
    pltpu.sync_copy(movies_hbm.at[wid], midx_v)
    pltpu.sync_copy(wb_hbm, w_v)
    pltpu.sync_copy(bb_hbm, b_v)

    urows2d = urows_v
    mrows2d = mrows_v
    copies = []
    for p in range(NP):
        copies.append(pltpu.async_copy(
            ut_hbm.at[uidx_v.at[p]],
            urows2d.at[pl.ds(p * IDX_CHUNK, IDX_CHUNK), :], sem))
        copies.append(pltpu.async_copy(
            mt_hbm.at[midx_v.at[p]],
            mrows2d.at[pl.ds(p * IDX_CHUNK, IDX_CHUNK), :], sem))
    for cp in copies:
        cp.wait()

    lane = lax.iota(jnp.int32, L)
    lane_e = lane * EMBED
    bias = b_v[...]

    def chunk(c, carry):
        rows = lane + c * L
        acc = bias
        for j in range(EMBED):
            jv = jnp.full((L,), j, jnp.int32)
            uj = plsc.load_gather(urows_v, [rows, jv])
            mj = plsc.load_gather(mrows_v, [rows, jv])
            acc = acc + uj * mj * w_v[j]
        res = 4.0 / (1.0 + jnp.exp(-acc)) + 1.0
        plsc.store_scatter(o_v, [rows], res)
        return carry

    lax.fori_loop(0, NCH, chunk, 0)
    pltpu.sync_copy(o_v, out_hbm.at[pl.ds(base, BPW)])


def kernel(users, movies, user_table, movie_table, W, b):
    users3 = users.astype(jnp.int32).reshape(NW, NP, IDX_CHUNK)
    movies3 = movies.astype(jnp.int32).reshape(NW, NP, IDX_CHUNK)
    wb = jnp.broadcast_to(W.reshape(EMBED, 1), (EMBED, L)).astype(jnp.float32)
    bb = jnp.broadcast_to(b.reshape(1), (L,)).astype(jnp.float32)

    mesh = plsc.VectorSubcoreMesh(core_axis_name="c", subcore_axis_name="s",
                                  num_cores=NC, num_subcores=NS)
    run = functools.partial(
        pl.kernel,
        out_type=jax.ShapeDtypeStruct((BATCH,), jnp.float32),
        mesh=mesh,
        scratch_types=[
            pltpu.VMEM((NP, IDX_CHUNK), jnp.int32),
            pltpu.VMEM((NP, IDX_CHUNK), jnp.int32),
            pltpu.VMEM((BPW, EMBED), jnp.float32),
            pltpu.VMEM((BPW, EMBED), jnp.float32),
            pltpu.VMEM((EMBED, L), jnp.float32),
            pltpu.VMEM((L,), jnp.float32),
            pltpu.VMEM((BPW,), jnp.float32),
            pltpu.SemaphoreType.DMA,
        ],
        compiler_params=pltpu.CompilerParams(needs_layout_passes=False,
                                             use_tc_tiling_on_sc=False),
    )(_gmf_body)
    return run(users3, movies3, user_table, movie_table, wb, bb)
